# Initial kernel scaffold; baseline (speedup 1.0000x reference)
import jax
import jax.numpy as jnp
from jax.experimental import pallas as pl


def kernel(x, edge_index, edge_attr, batch, W_np, b_np, bn0_g, bn0_b, W_ep, b_ep, Wf, bf, Ws, bs, bn_g, bn_b, W1, b1, W2, b2, W_sh, b_sh, W_bk, b_bk, W_yg, b_yg):
    raise NotImplementedError("write your pallas kernel here")


# speedup vs baseline: 1.8741x; 1.8741x over previous
import functools

import jax
import jax.numpy as jnp
from jax import lax
from jax.experimental import pallas as pl
from jax.experimental.pallas import tpu as pltpu
from jax.experimental.pallas import tpu_sc as plsc

N = 50000
E = 800000
G = 256
D = 64
DH = 32
NCONV = 3

NC = 2
NS = 16
NW = NC * NS

PER_W = E // NW
CG = 200
PER_S = E // NS
CS = 200

BN_BLK = 2000
BE_BLK = 4000

_MESH = plsc.VectorSubcoreMesh(
    core_axis_name="c", subcore_axis_name="s", num_cores=NC, num_subcores=NS)


@functools.partial(
    pl.kernel,
    out_type=[jax.ShapeDtypeStruct((E, D), jnp.float32),
              jax.ShapeDtypeStruct((E, D), jnp.float32)],
    mesh=_MESH,
    scratch_types=[pltpu.VMEM((CG,), jnp.int32),
                   pltpu.VMEM((CG,), jnp.int32),
                   pltpu.VMEM((CG, D), jnp.float32),
                   pltpu.VMEM((CG, D), jnp.float32),
                   pltpu.SemaphoreType.DMA,
                   pltpu.SemaphoreType.DMA],
    compiler_params=pltpu.CompilerParams(use_tc_tiling_on_sc=False),
)
def _gather2(h_hbm, dst_hbm, src_hbm, hd_hbm, hs_hbm,
             di_v, si_v, dr_v, sr_v, sem_d, sem_s):
    wid = lax.axis_index("s") * NC + lax.axis_index("c")
    base = wid * PER_W

    @pl.loop(0, PER_W, step=CG)
    def _(off):
        b = base + off
        pltpu.sync_copy(dst_hbm.at[pl.ds(b, CG)], di_v)
        pltpu.sync_copy(src_hbm.at[pl.ds(b, CG)], si_v)
        cp_d = pltpu.async_copy(h_hbm.at[di_v], dr_v, sem_d)
        cp_s = pltpu.async_copy(h_hbm.at[si_v], sr_v, sem_s)
        cp_d.wait()
        cp_s.wait()
        pltpu.sync_copy(dr_v, hd_hbm.at[pl.ds(b, CG)])
        pltpu.sync_copy(sr_v, hs_hbm.at[pl.ds(b, CG)])


@functools.partial(
    pl.kernel,
    out_type=jax.ShapeDtypeStruct((NC, N, DH), jnp.float32),
    mesh=_MESH,
    scratch_types=[pltpu.VMEM((CS,), jnp.int32),
                   pltpu.VMEM((CS, DH), jnp.float32),
                   pltpu.VMEM_SHARED((N, DH), jnp.float32)],
    compiler_params=pltpu.CompilerParams(use_tc_tiling_on_sc=False),
)
def _scatter_add(msg_hbm, dst_hbm, zero_hbm, agg_hbm, idx_v, m_v, acc_sh):
    c = lax.axis_index("c")
    s = lax.axis_index("s")

    @pl.when(s == 0)
    def _():
        pltpu.sync_copy(zero_hbm, acc_sh)

    plsc.subcore_barrier()
    base = s * PER_S

    @pl.loop(0, PER_S, step=CS)
    def _(off):
        b = base + off
        pltpu.sync_copy(dst_hbm.at[pl.ds(b, CS)], idx_v)
        pltpu.sync_copy(msg_hbm.at[c, pl.ds(b, CS)], m_v)
        pltpu.sync_copy(m_v, acc_sh.at[idx_v], add=True)

    plsc.subcore_barrier()
    rows = N // NS
    pltpu.sync_copy(acc_sh.at[pl.ds(s * rows, rows)],
                    agg_hbm.at[c, pl.ds(s * rows, rows)])


def _node_proj_body(x_ref, w_ref, b_ref, h_ref, st_ref):
    h = jax.nn.softplus(
        jnp.dot(x_ref[...], w_ref[...], preferred_element_type=jnp.float32)
        + b_ref[...])
    h_ref[...] = h
    contrib = jnp.concatenate(
        [jnp.sum(h, axis=0, keepdims=True),
         jnp.sum(h * h, axis=0, keepdims=True)], axis=0)

    @pl.when(pl.program_id(0) == 0)
    def _():
        st_ref[...] = contrib

    @pl.when(pl.program_id(0) != 0)
    def _():
        st_ref[...] += contrib


def _node_proj(x, w, b):
    nb = N // BN_BLK
    return pl.pallas_call(
        _node_proj_body,
        grid=(nb,),
        in_specs=[pl.BlockSpec((BN_BLK, x.shape[1]), lambda i: (i, 0)),
                  pl.BlockSpec(w.shape, lambda i: (0, 0)),
                  pl.BlockSpec(b.shape, lambda i: (0, 0))],
        out_specs=[pl.BlockSpec((BN_BLK, D), lambda i: (i, 0)),
                   pl.BlockSpec((2, D), lambda i: (0, 0))],
        out_shape=[jax.ShapeDtypeStruct((N, D), jnp.float32),
                   jax.ShapeDtypeStruct((2, D), jnp.float32)],
    )(x, w, b)


def _add_stats_body(h_ref, agg_ref, hn_ref, st_ref):
    a = agg_ref[...]
    hn = h_ref[...] + jnp.concatenate([a[0], a[1]], axis=-1)
    hn_ref[...] = hn
    contrib = jnp.concatenate(
        [jnp.sum(hn, axis=0, keepdims=True),
         jnp.sum(hn * hn, axis=0, keepdims=True)], axis=0)

    @pl.when(pl.program_id(0) == 0)
    def _():
        st_ref[...] = contrib

    @pl.when(pl.program_id(0) != 0)
    def _():
        st_ref[...] += contrib


def _add_stats(h, agg):
    nb = N // BN_BLK
    return pl.pallas_call(
        _add_stats_body,
        grid=(nb,),
        in_specs=[pl.BlockSpec((BN_BLK, D), lambda i: (i, 0)),
                  pl.BlockSpec((NC, BN_BLK, DH), lambda i: (0, i, 0))],
        out_specs=[pl.BlockSpec((BN_BLK, D), lambda i: (i, 0)),
                   pl.BlockSpec((2, D), lambda i: (0, 0))],
        out_shape=[jax.ShapeDtypeStruct((N, D), jnp.float32),
                   jax.ShapeDtypeStruct((2, D), jnp.float32)],
    )(h, agg)


def _affine_body(h_ref, sc_ref, sh_ref, o_ref):
    o_ref[...] = h_ref[...] * sc_ref[...] + sh_ref[...]


def _affine(h, scale, shift):
    nb = N // BN_BLK
    return pl.pallas_call(
        _affine_body,
        grid=(nb,),
        in_specs=[pl.BlockSpec((BN_BLK, D), lambda i: (i, 0)),
                  pl.BlockSpec((1, D), lambda i: (0, 0)),
                  pl.BlockSpec((1, D), lambda i: (0, 0))],
        out_specs=pl.BlockSpec((BN_BLK, D), lambda i: (i, 0)),
        out_shape=jax.ShapeDtypeStruct((N, D), jnp.float32),
    )(h, scale, shift)


def _edge_body(hd_ref, hs_ref, ea_ref, wep_ref, bep_ref,
               wf_ref, bf_ref, ws_ref, bs_ref, msg_ref):
    hd = hd_ref[...]
    hs = hs_ref[...]
    ea = ea_ref[...]
    wep = wep_ref[...]
    e = jax.nn.softplus(
        ea[:, 0:1] * wep[0:1, :] + ea[:, 1:2] * wep[1:2, :] + bep_ref[...])
    wf = wf_ref[...]
    ws = ws_ref[...]
    dot = lambda a, b: jnp.dot(a, b, preferred_element_type=jnp.float32)
    zf = (dot(hd, wf[0:D]) + dot(hs, wf[D:2 * D]) + dot(e, wf[2 * D:])
          + bf_ref[...])
    zs = (dot(hd, ws[0:D]) + dot(hs, ws[D:2 * D]) + dot(e, ws[2 * D:])
          + bs_ref[...])
    m = jax.nn.sigmoid(zf) * jax.nn.softplus(zs)
    msg_ref[0] = m[:, :DH]
    msg_ref[1] = m[:, DH:]


def _edge_compute(hd, hs, ea, wep, bep, wf_l, bf_l, ws_l, bs_l):
    nb = E // BE_BLK
    z = 2 * D + wep.shape[1]
    return pl.pallas_call(
        _edge_body,
        grid=(nb,),
        in_specs=[pl.BlockSpec((BE_BLK, D), lambda i: (i, 0)),
                  pl.BlockSpec((BE_BLK, D), lambda i: (i, 0)),
                  pl.BlockSpec((BE_BLK, ea.shape[1]), lambda i: (i, 0)),
                  pl.BlockSpec(wep.shape, lambda i: (0, 0)),
                  pl.BlockSpec(bep.shape, lambda i: (0, 0)),
                  pl.BlockSpec((z, D), lambda i: (0, 0)),
                  pl.BlockSpec((1, D), lambda i: (0, 0)),
                  pl.BlockSpec((z, D), lambda i: (0, 0)),
                  pl.BlockSpec((1, D), lambda i: (0, 0))],
        out_specs=pl.BlockSpec((NC, BE_BLK, DH), lambda i: (0, i, 0)),
        out_shape=jax.ShapeDtypeStruct((NC, E, DH), jnp.float32),
        compiler_params=pltpu.CompilerParams(
            dimension_semantics=("parallel",)),
    )(hd, hs, ea, wep, bep, wf_l, bf_l, ws_l, bs_l)


def _pool_body(h_ref, bat_ref, w1_ref, b1_ref, w2_ref, b2_ref,
               wo_ref, bo_ref, o_ref, acc_ref):
    bids = bat_ref[0, 0, :]
    oneh_t = (lax.broadcasted_iota(jnp.int32, (G, BN_BLK), 0)
              == bids[None, :]).astype(jnp.float32)
    h = h_ref[...]
    hb = jnp.concatenate([h, jnp.ones((BN_BLK, 1), jnp.float32)], axis=-1)
    contrib = jnp.dot(oneh_t, hb, preferred_element_type=jnp.float32)

    @pl.when(pl.program_id(0) == 0)
    def _():
        acc_ref[...] = contrib

    @pl.when(pl.program_id(0) != 0)
    def _():
        acc_ref[...] += contrib

    @pl.when(pl.program_id(0) == pl.num_programs(0) - 1)
    def _():
        acc = acc_ref[...]
        pooled = acc[:, :D] / jnp.maximum(acc[:, D:D + 1], 1.0)
        f = jax.nn.softplus(
            jnp.dot(pooled, w1_ref[...], preferred_element_type=jnp.float32)
            + b1_ref[...])
        f = jax.nn.softplus(
            jnp.dot(f, w2_ref[...], preferred_element_type=jnp.float32)
            + b2_ref[...])
        o_ref[...] = (jnp.dot(f, wo_ref[...],
                              preferred_element_type=jnp.float32)
                      + bo_ref[...])


def _pool_mlp(h, bat3, w1, b1, w2, b2, wo, bo):
    nb = N // BN_BLK
    hdim = w1.shape[1]
    return pl.pallas_call(
        _pool_body,
        grid=(nb,),
        in_specs=[pl.BlockSpec((BN_BLK, D), lambda i: (i, 0)),
                  pl.BlockSpec((1, 1, BN_BLK), lambda i: (i, 0, 0)),
                  pl.BlockSpec(w1.shape, lambda i: (0, 0)),
                  pl.BlockSpec(b1.shape, lambda i: (0, 0)),
                  pl.BlockSpec(w2.shape, lambda i: (0, 0)),
                  pl.BlockSpec(b2.shape, lambda i: (0, 0)),
                  pl.BlockSpec(wo.shape, lambda i: (0, 0)),
                  pl.BlockSpec(bo.shape, lambda i: (0, 0))],
        out_specs=pl.BlockSpec((G, wo.shape[1]), lambda i: (0, 0)),
        out_shape=jax.ShapeDtypeStruct((G, wo.shape[1]), jnp.float32),
        scratch_shapes=[pltpu.VMEM((G, D + 1), jnp.float32)],
    )(h, bat3, w1, b1, w2, b2, wo, bo)


def _bn_fold(stats, g, b):
    mu = stats[0] / N
    var = stats[1] / N - mu * mu
    scale = g / jnp.sqrt(var + 1e-5)
    shift = b - mu * scale
    return scale.reshape(1, D), shift.reshape(1, D)


def kernel(x, edge_index, edge_attr, batch, W_np, b_np, bn0_g, bn0_b,
           W_ep, b_ep, Wf, bf, Ws, bs, bn_g, bn_b, W1, b1, W2, b2,
           W_sh, b_sh, W_bk, b_bk, W_yg, b_yg):
    src = edge_index[0]
    dst = edge_index[1]

    h_pre, st = _node_proj(x, W_np, b_np.reshape(1, D))
    scale, shift = _bn_fold(st, bn0_g, bn0_b)
    h = _affine(h_pre, scale, shift)

    zero = jnp.zeros((N, DH), jnp.float32)
    for l in range(NCONV):
        hd, hs = _gather2(h, dst, src)
        msg = _edge_compute(hd, hs, edge_attr, W_ep,
                            b_ep.reshape(1, -1), Wf[l],
                            bf[l].reshape(1, D), Ws[l], bs[l].reshape(1, D))
        agg = _scatter_add(msg, dst, zero)
        hn, st = _add_stats(h, agg)
        scale, shift = _bn_fold(st, bn_g[l], bn_b[l])
        h = _affine(hn, scale, shift)

    wo = jnp.concatenate([W_sh, W_bk, W_yg], axis=1)
    bo = jnp.stack([b_sh[0], b_bk[0], b_yg[0]]).reshape(1, 3)
    out3 = _pool_mlp(h, batch.reshape(N // BN_BLK, 1, BN_BLK),
                     W1, b1.reshape(1, -1), W2, b2.reshape(1, -1), wo, bo)
    return out3[:, 0], out3[:, 1], out3[:, 2]


# speedup vs baseline: 2.1221x; 1.1323x over previous
import functools

import jax
import jax.numpy as jnp
from jax import lax
from jax.experimental import pallas as pl
from jax.experimental.pallas import tpu as pltpu
from jax.experimental.pallas import tpu_sc as plsc

N = 50000
E = 800000
G = 256
D = 64
DH = 32
NCONV = 3

NC = 2
NS = 16
NW = NC * NS

PER_W = E // NW
CG = 200
PER_S = E // NS
CS = 200

BN_BLK = 2000
BE_BLK = 4000

_MESH = plsc.VectorSubcoreMesh(
    core_axis_name="c", subcore_axis_name="s", num_cores=NC, num_subcores=NS)


@functools.partial(
    pl.kernel,
    out_type=[jax.ShapeDtypeStruct((E, 2 * D), jnp.float32),
              jax.ShapeDtypeStruct((E, 2 * D), jnp.float32)],
    mesh=_MESH,
    scratch_types=[pltpu.VMEM((CG,), jnp.int32),
                   pltpu.VMEM((CG,), jnp.int32),
                   pltpu.VMEM((CG, 2 * D), jnp.float32),
                   pltpu.VMEM((CG, 2 * D), jnp.float32),
                   pltpu.SemaphoreType.DMA,
                   pltpu.SemaphoreType.DMA],
    compiler_params=pltpu.CompilerParams(use_tc_tiling_on_sc=True),
)
def _gather2(td_hbm, ts_hbm, dst_hbm, src_hbm, gd_hbm, gs_hbm,
             di_v, si_v, dr_v, sr_v, sem_d, sem_s):
    wid = lax.axis_index("s") * NC + lax.axis_index("c")
    base = wid * PER_W

    @pl.loop(0, PER_W, step=CG)
    def _(off):
        b = base + off
        pltpu.sync_copy(dst_hbm.at[pl.ds(b, CG)], di_v)
        pltpu.sync_copy(src_hbm.at[pl.ds(b, CG)], si_v)
        cp_d = pltpu.async_copy(td_hbm.at[di_v], dr_v, sem_d)
        cp_s = pltpu.async_copy(ts_hbm.at[si_v], sr_v, sem_s)
        cp_d.wait()
        cp_s.wait()
        pltpu.sync_copy(dr_v, gd_hbm.at[pl.ds(b, CG)])
        pltpu.sync_copy(sr_v, gs_hbm.at[pl.ds(b, CG)])


@functools.partial(
    pl.kernel,
    out_type=jax.ShapeDtypeStruct((NC, N, DH), jnp.float32),
    mesh=_MESH,
    scratch_types=[pltpu.VMEM((CS,), jnp.int32),
                   pltpu.VMEM((CS, DH), jnp.float32),
                   pltpu.VMEM_SHARED((N, DH), jnp.float32)],
    compiler_params=pltpu.CompilerParams(use_tc_tiling_on_sc=False),
)
def _scatter_add(msg_hbm, dst_hbm, zero_hbm, agg_hbm, idx_v, m_v, acc_sh):
    c = lax.axis_index("c")
    s = lax.axis_index("s")

    @pl.when(s == 0)
    def _():
        pltpu.sync_copy(zero_hbm, acc_sh)

    plsc.subcore_barrier()
    base = s * PER_S

    @pl.loop(0, PER_S, step=CS)
    def _(off):
        b = base + off
        pltpu.sync_copy(dst_hbm.at[pl.ds(b, CS)], idx_v)
        pltpu.sync_copy(msg_hbm.at[c, pl.ds(b, CS)], m_v)
        pltpu.sync_copy(m_v, acc_sh.at[idx_v], add=True)

    plsc.subcore_barrier()
    rows = N // NS
    pltpu.sync_copy(acc_sh.at[pl.ds(s * rows, rows)],
                    agg_hbm.at[c, pl.ds(s * rows, rows)])


def _node_proj_body(x_ref, w_ref, b_ref, h_ref, st_ref):
    h = jax.nn.softplus(
        jnp.dot(x_ref[...], w_ref[...], preferred_element_type=jnp.float32)
        + b_ref[...])
    h_ref[...] = h
    contrib = jnp.concatenate(
        [jnp.sum(h, axis=0, keepdims=True),
         jnp.sum(h * h, axis=0, keepdims=True)], axis=0)

    @pl.when(pl.program_id(0) == 0)
    def _():
        st_ref[...] = contrib

    @pl.when(pl.program_id(0) != 0)
    def _():
        st_ref[...] += contrib


def _node_proj(x, w, b):
    nb = N // BN_BLK
    return pl.pallas_call(
        _node_proj_body,
        grid=(nb,),
        in_specs=[pl.BlockSpec((BN_BLK, x.shape[1]), lambda i: (i, 0)),
                  pl.BlockSpec(w.shape, lambda i: (0, 0)),
                  pl.BlockSpec(b.shape, lambda i: (0, 0))],
        out_specs=[pl.BlockSpec((BN_BLK, D), lambda i: (i, 0)),
                   pl.BlockSpec((2, D), lambda i: (0, 0))],
        out_shape=[jax.ShapeDtypeStruct((N, D), jnp.float32),
                   jax.ShapeDtypeStruct((2, D), jnp.float32)],
    )(x, w, b)


def _add_stats_body(h_ref, agg_ref, hn_ref, st_ref):
    a = agg_ref[...]
    hn = h_ref[...] + jnp.concatenate([a[0], a[1]], axis=-1)
    hn_ref[...] = hn
    contrib = jnp.concatenate(
        [jnp.sum(hn, axis=0, keepdims=True),
         jnp.sum(hn * hn, axis=0, keepdims=True)], axis=0)

    @pl.when(pl.program_id(0) == 0)
    def _():
        st_ref[...] = contrib

    @pl.when(pl.program_id(0) != 0)
    def _():
        st_ref[...] += contrib


def _add_stats(h, agg):
    nb = N // BN_BLK
    return pl.pallas_call(
        _add_stats_body,
        grid=(nb,),
        in_specs=[pl.BlockSpec((BN_BLK, D), lambda i: (i, 0)),
                  pl.BlockSpec((NC, BN_BLK, DH), lambda i: (0, i, 0))],
        out_specs=[pl.BlockSpec((BN_BLK, D), lambda i: (i, 0)),
                   pl.BlockSpec((2, D), lambda i: (0, 0))],
        out_shape=[jax.ShapeDtypeStruct((N, D), jnp.float32),
                   jax.ShapeDtypeStruct((2, D), jnp.float32)],
    )(h, agg)


def _affine_proj_body(h_ref, sc_ref, sh_ref, wd_ref, ws_ref,
                      o_ref, td_ref, ts_ref):
    hh = h_ref[...] * sc_ref[...] + sh_ref[...]
    o_ref[...] = hh
    td_ref[...] = jnp.dot(hh, wd_ref[...], preferred_element_type=jnp.float32)
    ts_ref[...] = jnp.dot(hh, ws_ref[...], preferred_element_type=jnp.float32)


def _affine_proj(h, scale, shift, wd, ws):
    nb = N // BN_BLK
    return pl.pallas_call(
        _affine_proj_body,
        grid=(nb,),
        in_specs=[pl.BlockSpec((BN_BLK, D), lambda i: (i, 0)),
                  pl.BlockSpec((1, D), lambda i: (0, 0)),
                  pl.BlockSpec((1, D), lambda i: (0, 0)),
                  pl.BlockSpec(wd.shape, lambda i: (0, 0)),
                  pl.BlockSpec(ws.shape, lambda i: (0, 0))],
        out_specs=[pl.BlockSpec((BN_BLK, D), lambda i: (i, 0)),
                   pl.BlockSpec((BN_BLK, 2 * D), lambda i: (i, 0)),
                   pl.BlockSpec((BN_BLK, 2 * D), lambda i: (i, 0))],
        out_shape=[jax.ShapeDtypeStruct((N, D), jnp.float32),
                   jax.ShapeDtypeStruct((N, 2 * D), jnp.float32),
                   jax.ShapeDtypeStruct((N, 2 * D), jnp.float32)],
        compiler_params=pltpu.CompilerParams(
            dimension_semantics=("parallel",)),
    )(h, scale, shift, wd, ws)


def _affine_body(h_ref, sc_ref, sh_ref, o_ref):
    o_ref[...] = h_ref[...] * sc_ref[...] + sh_ref[...]


def _affine(h, scale, shift):
    nb = N // BN_BLK
    return pl.pallas_call(
        _affine_body,
        grid=(nb,),
        in_specs=[pl.BlockSpec((BN_BLK, D), lambda i: (i, 0)),
                  pl.BlockSpec((1, D), lambda i: (0, 0)),
                  pl.BlockSpec((1, D), lambda i: (0, 0))],
        out_specs=pl.BlockSpec((BN_BLK, D), lambda i: (i, 0)),
        out_shape=jax.ShapeDtypeStruct((N, D), jnp.float32),
    )(h, scale, shift)


def _edge_body(gd_ref, gs_ref, ea_ref, wep_ref, bep_ref,
               wfe_ref, bf_ref, wse_ref, bs_ref, msg_ref):
    e = jax.nn.softplus(
        jnp.dot(ea_ref[...], wep_ref[...], preferred_element_type=jnp.float32)
        + bep_ref[...])
    g = gd_ref[...] + gs_ref[...]
    zf = (g[:, :D]
          + jnp.dot(e, wfe_ref[...], preferred_element_type=jnp.float32)
          + bf_ref[...])
    zs = (g[:, D:]
          + jnp.dot(e, wse_ref[...], preferred_element_type=jnp.float32)
          + bs_ref[...])
    m = jax.nn.sigmoid(zf) * jax.nn.softplus(zs)
    msg_ref[0] = m[:, :DH]
    msg_ref[1] = m[:, DH:]


def _edge_compute(gd, gs, ea, wep, bep, wfe, bf_l, wse, bs_l):
    nb = E // BE_BLK
    return pl.pallas_call(
        _edge_body,
        grid=(nb,),
        in_specs=[pl.BlockSpec((BE_BLK, 2 * D), lambda i: (i, 0)),
                  pl.BlockSpec((BE_BLK, 2 * D), lambda i: (i, 0)),
                  pl.BlockSpec((BE_BLK, 2), lambda i: (i, 0)),
                  pl.BlockSpec(wep.shape, lambda i: (0, 0)),
                  pl.BlockSpec(bep.shape, lambda i: (0, 0)),
                  pl.BlockSpec(wfe.shape, lambda i: (0, 0)),
                  pl.BlockSpec((1, D), lambda i: (0, 0)),
                  pl.BlockSpec(wse.shape, lambda i: (0, 0)),
                  pl.BlockSpec((1, D), lambda i: (0, 0))],
        out_specs=pl.BlockSpec((NC, BE_BLK, DH), lambda i: (0, i, 0)),
        out_shape=jax.ShapeDtypeStruct((NC, E, DH), jnp.float32),
        compiler_params=pltpu.CompilerParams(
            dimension_semantics=("parallel",)),
    )(gd, gs, ea, wep, bep, wfe, bf_l, wse, bs_l)


def _pool_body(h_ref, bat_ref, w1_ref, b1_ref, w2_ref, b2_ref,
               wo_ref, bo_ref, o_ref, acc_ref):
    bids = bat_ref[0, 0, :]
    oneh_t = (lax.broadcasted_iota(jnp.int32, (G, BN_BLK), 0)
              == bids[None, :]).astype(jnp.float32)
    h = h_ref[...]
    hb = jnp.concatenate([h, jnp.ones((BN_BLK, 1), jnp.float32)], axis=-1)
    contrib = jnp.dot(oneh_t, hb, preferred_element_type=jnp.float32)

    @pl.when(pl.program_id(0) == 0)
    def _():
        acc_ref[...] = contrib

    @pl.when(pl.program_id(0) != 0)
    def _():
        acc_ref[...] += contrib

    @pl.when(pl.program_id(0) == pl.num_programs(0) - 1)
    def _():
        acc = acc_ref[...]
        pooled = acc[:, :D] / jnp.maximum(acc[:, D:D + 1], 1.0)
        f = jax.nn.softplus(
            jnp.dot(pooled, w1_ref[...], preferred_element_type=jnp.float32)
            + b1_ref[...])
        f = jax.nn.softplus(
            jnp.dot(f, w2_ref[...], preferred_element_type=jnp.float32)
            + b2_ref[...])
        o_ref[...] = (jnp.dot(f, wo_ref[...],
                              preferred_element_type=jnp.float32)
                      + bo_ref[...])


def _pool_mlp(h, bat3, w1, b1, w2, b2, wo, bo):
    nb = N // BN_BLK
    hdim = w1.shape[1]
    return pl.pallas_call(
        _pool_body,
        grid=(nb,),
        in_specs=[pl.BlockSpec((BN_BLK, D), lambda i: (i, 0)),
                  pl.BlockSpec((1, 1, BN_BLK), lambda i: (i, 0, 0)),
                  pl.BlockSpec(w1.shape, lambda i: (0, 0)),
                  pl.BlockSpec(b1.shape, lambda i: (0, 0)),
                  pl.BlockSpec(w2.shape, lambda i: (0, 0)),
                  pl.BlockSpec(b2.shape, lambda i: (0, 0)),
                  pl.BlockSpec(wo.shape, lambda i: (0, 0)),
                  pl.BlockSpec(bo.shape, lambda i: (0, 0))],
        out_specs=pl.BlockSpec((G, wo.shape[1]), lambda i: (0, 0)),
        out_shape=jax.ShapeDtypeStruct((G, wo.shape[1]), jnp.float32),
        scratch_shapes=[pltpu.VMEM((G, D + 1), jnp.float32)],
    )(h, bat3, w1, b1, w2, b2, wo, bo)


def _bn_fold(stats, g, b):
    mu = stats[0] / N
    var = stats[1] / N - mu * mu
    scale = g / jnp.sqrt(var + 1e-5)
    shift = b - mu * scale
    return scale.reshape(1, D), shift.reshape(1, D)


def kernel(x, edge_index, edge_attr, batch, W_np, b_np, bn0_g, bn0_b,
           W_ep, b_ep, Wf, bf, Ws, bs, bn_g, bn_b, W1, b1, W2, b2,
           W_sh, b_sh, W_bk, b_bk, W_yg, b_yg):
    src = edge_index[0]
    dst = edge_index[1]

    hn, st = _node_proj(x, W_np, b_np.reshape(1, D))
    scale, shift = _bn_fold(st, bn0_g, bn0_b)

    zero = jnp.zeros((N, DH), jnp.float32)
    for l in range(NCONV):
        wd_l = jnp.concatenate([Wf[l][:D], Ws[l][:D]], axis=1)
        wsrc_l = jnp.concatenate([Wf[l][D:2 * D], Ws[l][D:2 * D]], axis=1)
        h, td, ts = _affine_proj(hn, scale, shift, wd_l, wsrc_l)
        gd, gs = _gather2(td, ts, dst, src)
        msg = _edge_compute(gd, gs, edge_attr, W_ep, b_ep.reshape(1, -1),
                            Wf[l][2 * D:], bf[l].reshape(1, D),
                            Ws[l][2 * D:], bs[l].reshape(1, D))
        agg = _scatter_add(msg, dst, zero)
        hn, st = _add_stats(h, agg)
        scale, shift = _bn_fold(st, bn_g[l], bn_b[l])
    h = _affine(hn, scale, shift)

    wo = jnp.concatenate([W_sh, W_bk, W_yg], axis=1)
    bo = jnp.stack([b_sh[0], b_bk[0], b_yg[0]]).reshape(1, 3)
    out3 = _pool_mlp(h, batch.reshape(N // BN_BLK, 1, BN_BLK),
                     W1, b1.reshape(1, -1), W2, b2.reshape(1, -1), wo, bo)
    return out3[:, 0], out3[:, 1], out3[:, 2]


# speedup vs baseline: 2.7091x; 1.2766x over previous
import functools

import jax
import jax.numpy as jnp
from jax import lax
from jax.experimental import pallas as pl
from jax.experimental.pallas import tpu as pltpu
from jax.experimental.pallas import tpu_sc as plsc

N = 50000
E = 800000
G = 256
D = 64
DH = 32
NCONV = 3

NC = 2
NS = 16
NW = NC * NS

PER_W = E // NW
CG = 200
PER_S = E // NS
CS = 200

BN_BLK = 2000
BE_BLK = 4000

_MESH = plsc.VectorSubcoreMesh(
    core_axis_name="c", subcore_axis_name="s", num_cores=NC, num_subcores=NS)


@functools.partial(
    pl.kernel,
    out_type=[jax.ShapeDtypeStruct((E, 2 * D), jnp.float32),
              jax.ShapeDtypeStruct((E, 2 * D), jnp.float32)],
    mesh=_MESH,
    scratch_types=[pltpu.VMEM((CG,), jnp.int32),
                   pltpu.VMEM((CG,), jnp.int32),
                   pltpu.VMEM((CG, 2 * D), jnp.float32),
                   pltpu.VMEM((CG, 2 * D), jnp.float32),
                   pltpu.SemaphoreType.DMA,
                   pltpu.SemaphoreType.DMA],
    compiler_params=pltpu.CompilerParams(use_tc_tiling_on_sc=True),
)
def _gather2(td_hbm, ts_hbm, dst_hbm, src_hbm, gd_hbm, gs_hbm,
             di_v, si_v, dr_v, sr_v, sem_d, sem_s):
    wid = lax.axis_index("s") * NC + lax.axis_index("c")
    base = wid * PER_W

    @pl.loop(0, PER_W, step=CG)
    def _(off):
        b = pl.multiple_of(base + off, 8)
        pltpu.sync_copy(dst_hbm.at[pl.ds(b, CG)], di_v)
        pltpu.sync_copy(src_hbm.at[pl.ds(b, CG)], si_v)
        cp_d = pltpu.async_copy(td_hbm.at[di_v], dr_v, sem_d)
        cp_s = pltpu.async_copy(ts_hbm.at[si_v], sr_v, sem_s)
        cp_d.wait()
        cp_s.wait()
        pltpu.sync_copy(dr_v, gd_hbm.at[pl.ds(b, CG)])
        pltpu.sync_copy(sr_v, gs_hbm.at[pl.ds(b, CG)])


@functools.partial(
    pl.kernel,
    out_type=jax.ShapeDtypeStruct((NC, N, DH), jnp.float32),
    mesh=_MESH,
    scratch_types=[pltpu.VMEM((CS,), jnp.int32),
                   pltpu.VMEM((CS, DH), jnp.float32),
                   pltpu.VMEM_SHARED((N, DH), jnp.float32)],
    compiler_params=pltpu.CompilerParams(use_tc_tiling_on_sc=False),
)
def _scatter_add(msg_hbm, dst_hbm, zero_hbm, agg_hbm, idx_v, m_v, acc_sh):
    c = lax.axis_index("c")
    s = lax.axis_index("s")

    @pl.when(s == 0)
    def _():
        pltpu.sync_copy(zero_hbm, acc_sh)

    plsc.subcore_barrier()
    base = s * PER_S

    @pl.loop(0, PER_S, step=CS)
    def _(off):
        b = pl.multiple_of(base + off, 8)
        pltpu.sync_copy(dst_hbm.at[pl.ds(b, CS)], idx_v)
        pltpu.sync_copy(msg_hbm.at[pl.ds(b, CS), pl.ds(DH * c, DH)], m_v)
        pltpu.sync_copy(m_v, acc_sh.at[idx_v], add=True)

    plsc.subcore_barrier()
    n_chunks = N // 400

    @pl.loop(0, (n_chunks + NS - 1) // NS)
    def _(k):
        chunk = s + k * NS

        @pl.when(chunk < n_chunks)
        def _():
            r0 = pl.multiple_of(chunk * 400, 8)
            pltpu.sync_copy(acc_sh.at[pl.ds(r0, 400)],
                            agg_hbm.at[c, pl.ds(r0, 400)])


def _node_proj_body(x_ref, w_ref, b_ref, h_ref, st_ref):
    h = jax.nn.softplus(
        jnp.dot(x_ref[...], w_ref[...], preferred_element_type=jnp.float32)
        + b_ref[...])
    h_ref[...] = h
    contrib = jnp.sum(h, axis=0, keepdims=True)

    @pl.when(pl.program_id(0) == 0)
    def _():
        st_ref[...] = contrib

    @pl.when(pl.program_id(0) != 0)
    def _():
        st_ref[...] += contrib


def _node_proj(x, w, b):
    nb = N // BN_BLK
    return pl.pallas_call(
        _node_proj_body,
        grid=(nb,),
        in_specs=[pl.BlockSpec((BN_BLK, x.shape[1]), lambda i: (i, 0)),
                  pl.BlockSpec(w.shape, lambda i: (0, 0)),
                  pl.BlockSpec(b.shape, lambda i: (0, 0))],
        out_specs=[pl.BlockSpec((BN_BLK, D), lambda i: (i, 0)),
                   pl.BlockSpec((1, D), lambda i: (0, 0))],
        out_shape=[jax.ShapeDtypeStruct((N, D), jnp.float32),
                   jax.ShapeDtypeStruct((1, D), jnp.float32)],
    )(x, w, b)


def _add_stats_body(h_ref, agg_ref, hn_ref, st_ref):
    a = agg_ref[...]
    hn = h_ref[...] + jnp.concatenate([a[0], a[1]], axis=-1)
    hn_ref[...] = hn
    contrib = jnp.sum(hn, axis=0, keepdims=True)

    @pl.when(pl.program_id(0) == 0)
    def _():
        st_ref[...] = contrib

    @pl.when(pl.program_id(0) != 0)
    def _():
        st_ref[...] += contrib


def _add_stats(h, agg):
    nb = N // BN_BLK
    return pl.pallas_call(
        _add_stats_body,
        grid=(nb,),
        in_specs=[pl.BlockSpec((BN_BLK, D), lambda i: (i, 0)),
                  pl.BlockSpec((NC, BN_BLK, DH), lambda i: (0, i, 0))],
        out_specs=[pl.BlockSpec((BN_BLK, D), lambda i: (i, 0)),
                   pl.BlockSpec((1, D), lambda i: (0, 0))],
        out_shape=[jax.ShapeDtypeStruct((N, D), jnp.float32),
                   jax.ShapeDtypeStruct((1, D), jnp.float32)],
    )(h, agg)


def _var_body(h_ref, mu_ref, st_ref):
    d = h_ref[...] - mu_ref[...]
    contrib = jnp.sum(d * d, axis=0, keepdims=True)

    @pl.when(pl.program_id(0) == 0)
    def _():
        st_ref[...] = contrib

    @pl.when(pl.program_id(0) != 0)
    def _():
        st_ref[...] += contrib


def _var_pass(h, mu):
    nb = N // BN_BLK
    return pl.pallas_call(
        _var_body,
        grid=(nb,),
        in_specs=[pl.BlockSpec((BN_BLK, D), lambda i: (i, 0)),
                  pl.BlockSpec((1, D), lambda i: (0, 0))],
        out_specs=pl.BlockSpec((1, D), lambda i: (0, 0)),
        out_shape=jax.ShapeDtypeStruct((1, D), jnp.float32),
    )(h, mu)


def _affine_proj_body(h_ref, g_ref, b_ref, mu_ref, den_ref, wd_ref, ws_ref,
                      o_ref, td_ref, ts_ref):
    hh = g_ref[...] * (h_ref[...] - mu_ref[...]) / den_ref[...] + b_ref[...]
    o_ref[...] = hh
    td_ref[...] = jnp.dot(hh, wd_ref[...], preferred_element_type=jnp.float32)
    ts_ref[...] = jnp.dot(hh, ws_ref[...], preferred_element_type=jnp.float32)


def _affine_proj(h, g, b, mu, den, wd, ws):
    nb = N // BN_BLK
    return pl.pallas_call(
        _affine_proj_body,
        grid=(nb,),
        in_specs=[pl.BlockSpec((BN_BLK, D), lambda i: (i, 0)),
                  pl.BlockSpec((1, D), lambda i: (0, 0)),
                  pl.BlockSpec((1, D), lambda i: (0, 0)),
                  pl.BlockSpec((1, D), lambda i: (0, 0)),
                  pl.BlockSpec((1, D), lambda i: (0, 0)),
                  pl.BlockSpec(wd.shape, lambda i: (0, 0)),
                  pl.BlockSpec(ws.shape, lambda i: (0, 0))],
        out_specs=[pl.BlockSpec((BN_BLK, D), lambda i: (i, 0)),
                   pl.BlockSpec((BN_BLK, 2 * D), lambda i: (i, 0)),
                   pl.BlockSpec((BN_BLK, 2 * D), lambda i: (i, 0))],
        out_shape=[jax.ShapeDtypeStruct((N, D), jnp.float32),
                   jax.ShapeDtypeStruct((N, 2 * D), jnp.float32),
                   jax.ShapeDtypeStruct((N, 2 * D), jnp.float32)],
        compiler_params=pltpu.CompilerParams(
            dimension_semantics=("parallel",)),
    )(h, g, b, mu, den, wd, ws)


def _affine_body(h_ref, g_ref, b_ref, mu_ref, den_ref, o_ref):
    o_ref[...] = (g_ref[...] * (h_ref[...] - mu_ref[...]) / den_ref[...]
                  + b_ref[...])


def _affine(h, g, b, mu, den):
    nb = N // BN_BLK
    return pl.pallas_call(
        _affine_body,
        grid=(nb,),
        in_specs=[pl.BlockSpec((BN_BLK, D), lambda i: (i, 0)),
                  pl.BlockSpec((1, D), lambda i: (0, 0)),
                  pl.BlockSpec((1, D), lambda i: (0, 0)),
                  pl.BlockSpec((1, D), lambda i: (0, 0)),
                  pl.BlockSpec((1, D), lambda i: (0, 0))],
        out_specs=pl.BlockSpec((BN_BLK, D), lambda i: (i, 0)),
        out_shape=jax.ShapeDtypeStruct((N, D), jnp.float32),
    )(h, g, b, mu, den)


def _edge_body(gd_ref, gs_ref, ea_ref, wep_ref, bep_ref,
               wfe_ref, bf_ref, wse_ref, bs_ref, msg_ref):
    e = jax.nn.softplus(
        jnp.dot(ea_ref[...], wep_ref[...], preferred_element_type=jnp.float32)
        + bep_ref[...])
    g = gd_ref[...] + gs_ref[...]
    zf = (g[:, :D]
          + jnp.dot(e, wfe_ref[...], preferred_element_type=jnp.float32)
          + bf_ref[...])
    zs = (g[:, D:]
          + jnp.dot(e, wse_ref[...], preferred_element_type=jnp.float32)
          + bs_ref[...])
    m = jax.nn.sigmoid(zf) * jax.nn.softplus(zs)
    msg_ref[...] = jnp.concatenate([m, m], axis=-1)


def _edge_compute(gd, gs, ea, wep, bep, wfe, bf_l, wse, bs_l):
    nb = E // BE_BLK
    return pl.pallas_call(
        _edge_body,
        grid=(nb,),
        in_specs=[pl.BlockSpec((BE_BLK, 2 * D), lambda i: (i, 0)),
                  pl.BlockSpec((BE_BLK, 2 * D), lambda i: (i, 0)),
                  pl.BlockSpec((BE_BLK, 2), lambda i: (i, 0)),
                  pl.BlockSpec(wep.shape, lambda i: (0, 0)),
                  pl.BlockSpec(bep.shape, lambda i: (0, 0)),
                  pl.BlockSpec(wfe.shape, lambda i: (0, 0)),
                  pl.BlockSpec((1, D), lambda i: (0, 0)),
                  pl.BlockSpec(wse.shape, lambda i: (0, 0)),
                  pl.BlockSpec((1, D), lambda i: (0, 0))],
        out_specs=pl.BlockSpec((BE_BLK, 2 * D), lambda i: (i, 0)),
        out_shape=jax.ShapeDtypeStruct((E, 2 * D), jnp.float32),
        compiler_params=pltpu.CompilerParams(
            dimension_semantics=("parallel",)),
    )(gd, gs, ea, wep, bep, wfe, bf_l, wse, bs_l)


def _pool_body(h_ref, bat_ref, w1_ref, b1_ref, w2_ref, b2_ref,
               wo_ref, bo_ref, o_ref, acc_ref):
    bids = bat_ref[0, 0, :]
    oneh_t = (lax.broadcasted_iota(jnp.int32, (G, BN_BLK), 0)
              == bids[None, :]).astype(jnp.float32)
    h = h_ref[...]
    hb = jnp.concatenate([h, jnp.ones((BN_BLK, 1), jnp.float32)], axis=-1)
    contrib = jnp.dot(oneh_t, hb, preferred_element_type=jnp.float32,
                      precision=lax.Precision.HIGHEST)

    @pl.when(pl.program_id(0) == 0)
    def _():
        acc_ref[...] = contrib

    @pl.when(pl.program_id(0) != 0)
    def _():
        acc_ref[...] += contrib

    @pl.when(pl.program_id(0) == pl.num_programs(0) - 1)
    def _():
        acc = acc_ref[...]
        pooled = acc[:, :D] / jnp.maximum(acc[:, D:D + 1], 1.0)
        f = jax.nn.softplus(
            jnp.dot(pooled, w1_ref[...], preferred_element_type=jnp.float32)
            + b1_ref[...])
        f = jax.nn.softplus(
            jnp.dot(f, w2_ref[...], preferred_element_type=jnp.float32)
            + b2_ref[...])
        o_ref[...] = (jnp.dot(f, wo_ref[...],
                              preferred_element_type=jnp.float32)
                      + bo_ref[...])


def _pool_mlp(h, bat3, w1, b1, w2, b2, wo, bo):
    nb = N // BN_BLK
    hdim = w1.shape[1]
    return pl.pallas_call(
        _pool_body,
        grid=(nb,),
        in_specs=[pl.BlockSpec((BN_BLK, D), lambda i: (i, 0)),
                  pl.BlockSpec((1, 1, BN_BLK), lambda i: (i, 0, 0)),
                  pl.BlockSpec(w1.shape, lambda i: (0, 0)),
                  pl.BlockSpec(b1.shape, lambda i: (0, 0)),
                  pl.BlockSpec(w2.shape, lambda i: (0, 0)),
                  pl.BlockSpec(b2.shape, lambda i: (0, 0)),
                  pl.BlockSpec(wo.shape, lambda i: (0, 0)),
                  pl.BlockSpec(bo.shape, lambda i: (0, 0))],
        out_specs=pl.BlockSpec((G, wo.shape[1]), lambda i: (0, 0)),
        out_shape=jax.ShapeDtypeStruct((G, wo.shape[1]), jnp.float32),
        scratch_shapes=[pltpu.VMEM((G, D + 1), jnp.float32)],
    )(h, bat3, w1, b1, w2, b2, wo, bo)


def _bn_stats(h, s1):
    mu = s1 / N
    var = _var_pass(h, mu) / N
    den = jnp.sqrt(var + 1e-5)
    return mu, den


def kernel(x, edge_index, edge_attr, batch, W_np, b_np, bn0_g, bn0_b,
           W_ep, b_ep, Wf, bf, Ws, bs, bn_g, bn_b, W1, b1, W2, b2,
           W_sh, b_sh, W_bk, b_bk, W_yg, b_yg):
    src = edge_index[0]
    dst = edge_index[1]

    hn, s1 = _node_proj(x, W_np, b_np.reshape(1, D))
    mu, den = _bn_stats(hn, s1)
    g_l, b_l = bn0_g.reshape(1, D), bn0_b.reshape(1, D)

    zero = jnp.zeros((N, DH), jnp.float32)
    for l in range(NCONV):
        wd_l = jnp.concatenate([Wf[l][:D], Ws[l][:D]], axis=1)
        wsrc_l = jnp.concatenate([Wf[l][D:2 * D], Ws[l][D:2 * D]], axis=1)
        h, td, ts = _affine_proj(hn, g_l, b_l, mu, den, wd_l, wsrc_l)
        gd, gs = _gather2(td, ts, dst, src)
        msg = _edge_compute(gd, gs, edge_attr, W_ep, b_ep.reshape(1, -1),
                            Wf[l][2 * D:], bf[l].reshape(1, D),
                            Ws[l][2 * D:], bs[l].reshape(1, D))
        agg = _scatter_add(msg, dst, zero)
        hn, s1 = _add_stats(h, agg)
        mu, den = _bn_stats(hn, s1)
        g_l, b_l = bn_g[l].reshape(1, D), bn_b[l].reshape(1, D)
    h = _affine(hn, g_l, b_l, mu, den)

    wo = jnp.concatenate([W_sh, W_bk, W_yg], axis=1)
    bo = jnp.stack([b_sh[0], b_bk[0], b_yg[0]]).reshape(1, 3)
    out3 = _pool_mlp(h, batch.reshape(N // BN_BLK, 1, BN_BLK),
                     W1, b1.reshape(1, -1), W2, b2.reshape(1, -1), wo, bo)
    return out3[:, 0], out3[:, 1], out3[:, 2]


# speedup vs baseline: 3.2919x; 1.2151x over previous
import functools

import jax
import jax.numpy as jnp
from jax import lax
from jax.experimental import pallas as pl
from jax.experimental.pallas import tpu as pltpu
from jax.experimental.pallas import tpu_sc as plsc

N = 50000
E = 800000
G = 256
D = 64
DH = 32
NCONV = 3

NC = 2
NS = 16
NW = NC * NS

PER_W = E // NW
CG = 200
PER_S = E // NS
CS = 200

BN_BLK = 2000
BE_BLK = 4000

_MESH = plsc.VectorSubcoreMesh(
    core_axis_name="c", subcore_axis_name="s", num_cores=NC, num_subcores=NS)


@functools.partial(
    pl.kernel,
    out_type=[jax.ShapeDtypeStruct((E, 2 * D), jnp.float32),
              jax.ShapeDtypeStruct((E, 2 * D), jnp.float32)],
    mesh=_MESH,
    scratch_types=[pltpu.VMEM((CG,), jnp.int32),
                   pltpu.VMEM((CG,), jnp.int32),
                   pltpu.VMEM((CG,), jnp.int32),
                   pltpu.VMEM((CG,), jnp.int32),
                   pltpu.VMEM((CG, 2 * D), jnp.float32),
                   pltpu.VMEM((CG, 2 * D), jnp.float32),
                   pltpu.VMEM((CG, 2 * D), jnp.float32),
                   pltpu.VMEM((CG, 2 * D), jnp.float32),
                   pltpu.SemaphoreType.DMA((NC,)),
                   pltpu.SemaphoreType.DMA((NC,))],
    compiler_params=pltpu.CompilerParams(use_tc_tiling_on_sc=True),
)
def _gather2(td_hbm, ts_hbm, dst_hbm, src_hbm, gd_hbm, gs_hbm,
             di_v0, di_v1, si_v0, si_v1, dr_v0, dr_v1, sr_v0, sr_v1,
             sem_d, sem_s):
    di_vs, si_vs = (di_v0, di_v1), (si_v0, si_v1)
    dr_vs, sr_vs = (dr_v0, dr_v1), (sr_v0, sr_v1)
    wid = lax.axis_index("s") * NC + lax.axis_index("c")
    base = wid * PER_W

    def fire(k, slot):
        b = pl.multiple_of(base + k * CG, 8)
        pltpu.sync_copy(dst_hbm.at[pl.ds(b, CG)], di_vs[slot])
        pltpu.sync_copy(src_hbm.at[pl.ds(b, CG)], si_vs[slot])
        pltpu.async_copy(td_hbm.at[di_vs[slot]], dr_vs[slot],
                         sem_d.at[slot])
        pltpu.async_copy(ts_hbm.at[si_vs[slot]], sr_vs[slot],
                         sem_s.at[slot])

    def drain(k, slot):
        b = pl.multiple_of(base + k * CG, 8)
        pltpu.make_async_copy(td_hbm.at[di_vs[slot]], dr_vs[slot],
                              sem_d.at[slot]).wait()
        pltpu.make_async_copy(ts_hbm.at[si_vs[slot]], sr_vs[slot],
                              sem_s.at[slot]).wait()
        pltpu.sync_copy(dr_vs[slot], gd_hbm.at[pl.ds(b, CG)])
        pltpu.sync_copy(sr_vs[slot], gs_hbm.at[pl.ds(b, CG)])

    fire(0, 0)

    @pl.loop(0, (PER_W // CG - 1) // 2)
    def _(it):
        k0 = 1 + 2 * it
        fire(k0, 1)
        drain(k0 - 1, 0)
        fire(k0 + 1, 0)
        drain(k0, 1)

    drain(PER_W // CG - 1, 0)


@functools.partial(
    pl.kernel,
    out_type=jax.ShapeDtypeStruct((NC, N, DH), jnp.float32),
    mesh=_MESH,
    scratch_types=[pltpu.VMEM((CS,), jnp.int32),
                   pltpu.VMEM((CS,), jnp.int32),
                   pltpu.VMEM((CS, DH), jnp.float32),
                   pltpu.VMEM((CS, DH), jnp.float32),
                   pltpu.VMEM_SHARED((N, DH), jnp.float32),
                   pltpu.SemaphoreType.DMA((NC,)),
                   pltpu.SemaphoreType.DMA((NC,))],
    compiler_params=pltpu.CompilerParams(use_tc_tiling_on_sc=False),
)
def _scatter_add(msg_hbm, dst_hbm, zero_hbm, agg_hbm, idx_v0, idx_v1,
                 m_v0, m_v1, acc_sh, sem_i, sem_m):
    idx_vs = (idx_v0, idx_v1)
    m_vs = (m_v0, m_v1)
    c = lax.axis_index("c")
    s = lax.axis_index("s")

    @pl.when(s == 0)
    def _():
        pltpu.sync_copy(zero_hbm, acc_sh)

    plsc.subcore_barrier()
    base = s * PER_S

    def fire(k, slot):
        b = pl.multiple_of(base + k * CS, 8)
        pltpu.async_copy(dst_hbm.at[pl.ds(b, CS)], idx_vs[slot],
                         sem_i.at[slot])
        pltpu.async_copy(msg_hbm.at[pl.ds(b, CS), pl.ds(DH * c, DH)],
                         m_vs[slot], sem_m.at[slot])

    def add(k, slot):
        b = pl.multiple_of(base + k * CS, 8)
        pltpu.make_async_copy(dst_hbm.at[pl.ds(b, CS)], idx_vs[slot],
                              sem_i.at[slot]).wait()
        pltpu.make_async_copy(msg_hbm.at[pl.ds(b, CS), pl.ds(DH * c, DH)],
                              m_vs[slot], sem_m.at[slot]).wait()
        pltpu.sync_copy(m_vs[slot], acc_sh.at[idx_vs[slot]], add=True)

    fire(0, 0)

    @pl.loop(0, PER_S // CS // 2)
    def _(it):
        k0 = 2 * it

        @pl.when(k0 + 1 < PER_S // CS)
        def _():
            fire(k0 + 1, 1)

        add(k0, 0)

        @pl.when(k0 + 2 < PER_S // CS)
        def _():
            fire(k0 + 2, 0)

        @pl.when(k0 + 1 < PER_S // CS)
        def _():
            add(k0 + 1, 1)

    plsc.subcore_barrier()
    n_chunks = N // 400

    @pl.loop(0, (n_chunks + NS - 1) // NS)
    def _(k):
        chunk = s + k * NS

        @pl.when(chunk < n_chunks)
        def _():
            r0 = pl.multiple_of(chunk * 400, 8)
            pltpu.sync_copy(acc_sh.at[pl.ds(r0, 400)],
                            agg_hbm.at[c, pl.ds(r0, 400)])


def _node_proj_body(x_ref, w_ref, b_ref, h_ref, st_ref):
    h = jax.nn.softplus(
        jnp.dot(x_ref[...], w_ref[...], preferred_element_type=jnp.float32)
        + b_ref[...])
    h_ref[...] = h
    contrib = jnp.sum(h, axis=0, keepdims=True)

    @pl.when(pl.program_id(0) == 0)
    def _():
        st_ref[...] = contrib

    @pl.when(pl.program_id(0) != 0)
    def _():
        st_ref[...] += contrib


def _node_proj(x, w, b):
    nb = N // BN_BLK
    return pl.pallas_call(
        _node_proj_body,
        grid=(nb,),
        in_specs=[pl.BlockSpec((BN_BLK, x.shape[1]), lambda i: (i, 0)),
                  pl.BlockSpec(w.shape, lambda i: (0, 0)),
                  pl.BlockSpec(b.shape, lambda i: (0, 0))],
        out_specs=[pl.BlockSpec((BN_BLK, D), lambda i: (i, 0)),
                   pl.BlockSpec((1, D), lambda i: (0, 0))],
        out_shape=[jax.ShapeDtypeStruct((N, D), jnp.float32),
                   jax.ShapeDtypeStruct((1, D), jnp.float32)],
    )(x, w, b)


def _add_stats_body(h_ref, agg_ref, hn_ref, st_ref):
    a = agg_ref[...]
    hn = h_ref[...] + jnp.concatenate([a[0], a[1]], axis=-1)
    hn_ref[...] = hn
    contrib = jnp.sum(hn, axis=0, keepdims=True)

    @pl.when(pl.program_id(0) == 0)
    def _():
        st_ref[...] = contrib

    @pl.when(pl.program_id(0) != 0)
    def _():
        st_ref[...] += contrib


def _add_stats(h, agg):
    nb = N // BN_BLK
    return pl.pallas_call(
        _add_stats_body,
        grid=(nb,),
        in_specs=[pl.BlockSpec((BN_BLK, D), lambda i: (i, 0)),
                  pl.BlockSpec((NC, BN_BLK, DH), lambda i: (0, i, 0))],
        out_specs=[pl.BlockSpec((BN_BLK, D), lambda i: (i, 0)),
                   pl.BlockSpec((1, D), lambda i: (0, 0))],
        out_shape=[jax.ShapeDtypeStruct((N, D), jnp.float32),
                   jax.ShapeDtypeStruct((1, D), jnp.float32)],
    )(h, agg)


def _var_body(h_ref, mu_ref, st_ref):
    d = h_ref[...] - mu_ref[...]
    contrib = jnp.sum(d * d, axis=0, keepdims=True)

    @pl.when(pl.program_id(0) == 0)
    def _():
        st_ref[...] = contrib

    @pl.when(pl.program_id(0) != 0)
    def _():
        st_ref[...] += contrib


def _var_pass(h, mu):
    nb = N // BN_BLK
    return pl.pallas_call(
        _var_body,
        grid=(nb,),
        in_specs=[pl.BlockSpec((BN_BLK, D), lambda i: (i, 0)),
                  pl.BlockSpec((1, D), lambda i: (0, 0))],
        out_specs=pl.BlockSpec((1, D), lambda i: (0, 0)),
        out_shape=jax.ShapeDtypeStruct((1, D), jnp.float32),
    )(h, mu)


def _affine_proj_body(h_ref, g_ref, b_ref, mu_ref, den_ref, wd_ref, ws_ref,
                      o_ref, td_ref, ts_ref):
    hh = g_ref[...] * (h_ref[...] - mu_ref[...]) / den_ref[...] + b_ref[...]
    o_ref[...] = hh
    td_ref[...] = jnp.dot(hh, wd_ref[...], preferred_element_type=jnp.float32)
    ts_ref[...] = jnp.dot(hh, ws_ref[...], preferred_element_type=jnp.float32)


def _affine_proj(h, g, b, mu, den, wd, ws):
    nb = N // BN_BLK
    return pl.pallas_call(
        _affine_proj_body,
        grid=(nb,),
        in_specs=[pl.BlockSpec((BN_BLK, D), lambda i: (i, 0)),
                  pl.BlockSpec((1, D), lambda i: (0, 0)),
                  pl.BlockSpec((1, D), lambda i: (0, 0)),
                  pl.BlockSpec((1, D), lambda i: (0, 0)),
                  pl.BlockSpec((1, D), lambda i: (0, 0)),
                  pl.BlockSpec(wd.shape, lambda i: (0, 0)),
                  pl.BlockSpec(ws.shape, lambda i: (0, 0))],
        out_specs=[pl.BlockSpec((BN_BLK, D), lambda i: (i, 0)),
                   pl.BlockSpec((BN_BLK, 2 * D), lambda i: (i, 0)),
                   pl.BlockSpec((BN_BLK, 2 * D), lambda i: (i, 0))],
        out_shape=[jax.ShapeDtypeStruct((N, D), jnp.float32),
                   jax.ShapeDtypeStruct((N, 2 * D), jnp.float32),
                   jax.ShapeDtypeStruct((N, 2 * D), jnp.float32)],
        compiler_params=pltpu.CompilerParams(
            dimension_semantics=("parallel",)),
    )(h, g, b, mu, den, wd, ws)


def _affine_body(h_ref, g_ref, b_ref, mu_ref, den_ref, o_ref):
    o_ref[...] = (g_ref[...] * (h_ref[...] - mu_ref[...]) / den_ref[...]
                  + b_ref[...])


def _affine(h, g, b, mu, den):
    nb = N // BN_BLK
    return pl.pallas_call(
        _affine_body,
        grid=(nb,),
        in_specs=[pl.BlockSpec((BN_BLK, D), lambda i: (i, 0)),
                  pl.BlockSpec((1, D), lambda i: (0, 0)),
                  pl.BlockSpec((1, D), lambda i: (0, 0)),
                  pl.BlockSpec((1, D), lambda i: (0, 0)),
                  pl.BlockSpec((1, D), lambda i: (0, 0))],
        out_specs=pl.BlockSpec((BN_BLK, D), lambda i: (i, 0)),
        out_shape=jax.ShapeDtypeStruct((N, D), jnp.float32),
    )(h, g, b, mu, den)


def _edge_body(gd_ref, gs_ref, ea_ref, wep_ref, bep_ref,
               wfe_ref, bf_ref, wse_ref, bs_ref, msg_ref):
    e = jax.nn.softplus(
        jnp.dot(ea_ref[...], wep_ref[...], preferred_element_type=jnp.float32)
        + bep_ref[...])
    g = gd_ref[...] + gs_ref[...]
    zf = (g[:, :D]
          + jnp.dot(e, wfe_ref[...], preferred_element_type=jnp.float32)
          + bf_ref[...])
    zs = (g[:, D:]
          + jnp.dot(e, wse_ref[...], preferred_element_type=jnp.float32)
          + bs_ref[...])
    m = jax.nn.sigmoid(zf) * jax.nn.softplus(zs)
    msg_ref[...] = jnp.concatenate([m, m], axis=-1)


def _edge_compute(gd, gs, ea, wep, bep, wfe, bf_l, wse, bs_l):
    nb = E // BE_BLK
    return pl.pallas_call(
        _edge_body,
        grid=(nb,),
        in_specs=[pl.BlockSpec((BE_BLK, 2 * D), lambda i: (i, 0)),
                  pl.BlockSpec((BE_BLK, 2 * D), lambda i: (i, 0)),
                  pl.BlockSpec((BE_BLK, 2), lambda i: (i, 0)),
                  pl.BlockSpec(wep.shape, lambda i: (0, 0)),
                  pl.BlockSpec(bep.shape, lambda i: (0, 0)),
                  pl.BlockSpec(wfe.shape, lambda i: (0, 0)),
                  pl.BlockSpec((1, D), lambda i: (0, 0)),
                  pl.BlockSpec(wse.shape, lambda i: (0, 0)),
                  pl.BlockSpec((1, D), lambda i: (0, 0))],
        out_specs=pl.BlockSpec((BE_BLK, 2 * D), lambda i: (i, 0)),
        out_shape=jax.ShapeDtypeStruct((E, 2 * D), jnp.float32),
        compiler_params=pltpu.CompilerParams(
            dimension_semantics=("parallel",)),
    )(gd, gs, ea, wep, bep, wfe, bf_l, wse, bs_l)


def _pool_body(h_ref, bat_ref, w1_ref, b1_ref, w2_ref, b2_ref,
               wo_ref, bo_ref, o_ref, acc_ref):
    bids = bat_ref[0, 0, :]
    oneh_t = (lax.broadcasted_iota(jnp.int32, (G, BN_BLK), 0)
              == bids[None, :]).astype(jnp.float32)
    h = h_ref[...]
    hb = jnp.concatenate([h, jnp.ones((BN_BLK, 1), jnp.float32)], axis=-1)
    contrib = jnp.dot(oneh_t, hb, preferred_element_type=jnp.float32,
                      precision=lax.Precision.HIGHEST)

    @pl.when(pl.program_id(0) == 0)
    def _():
        acc_ref[...] = contrib

    @pl.when(pl.program_id(0) != 0)
    def _():
        acc_ref[...] += contrib

    @pl.when(pl.program_id(0) == pl.num_programs(0) - 1)
    def _():
        acc = acc_ref[...]
        pooled = acc[:, :D] / jnp.maximum(acc[:, D:D + 1], 1.0)
        f = jax.nn.softplus(
            jnp.dot(pooled, w1_ref[...], preferred_element_type=jnp.float32)
            + b1_ref[...])
        f = jax.nn.softplus(
            jnp.dot(f, w2_ref[...], preferred_element_type=jnp.float32)
            + b2_ref[...])
        o_ref[...] = (jnp.dot(f, wo_ref[...],
                              preferred_element_type=jnp.float32)
                      + bo_ref[...])


def _pool_mlp(h, bat3, w1, b1, w2, b2, wo, bo):
    nb = N // BN_BLK
    hdim = w1.shape[1]
    return pl.pallas_call(
        _pool_body,
        grid=(nb,),
        in_specs=[pl.BlockSpec((BN_BLK, D), lambda i: (i, 0)),
                  pl.BlockSpec((1, 1, BN_BLK), lambda i: (i, 0, 0)),
                  pl.BlockSpec(w1.shape, lambda i: (0, 0)),
                  pl.BlockSpec(b1.shape, lambda i: (0, 0)),
                  pl.BlockSpec(w2.shape, lambda i: (0, 0)),
                  pl.BlockSpec(b2.shape, lambda i: (0, 0)),
                  pl.BlockSpec(wo.shape, lambda i: (0, 0)),
                  pl.BlockSpec(bo.shape, lambda i: (0, 0))],
        out_specs=pl.BlockSpec((G, wo.shape[1]), lambda i: (0, 0)),
        out_shape=jax.ShapeDtypeStruct((G, wo.shape[1]), jnp.float32),
        scratch_shapes=[pltpu.VMEM((G, D + 1), jnp.float32)],
    )(h, bat3, w1, b1, w2, b2, wo, bo)


def _bn_stats(h, s1):
    mu = s1 / N
    var = _var_pass(h, mu) / N
    den = jnp.sqrt(var + 1e-5)
    return mu, den


def kernel(x, edge_index, edge_attr, batch, W_np, b_np, bn0_g, bn0_b,
           W_ep, b_ep, Wf, bf, Ws, bs, bn_g, bn_b, W1, b1, W2, b2,
           W_sh, b_sh, W_bk, b_bk, W_yg, b_yg):
    src = edge_index[0]
    dst = edge_index[1]

    hn, s1 = _node_proj(x, W_np, b_np.reshape(1, D))
    mu, den = _bn_stats(hn, s1)
    g_l, b_l = bn0_g.reshape(1, D), bn0_b.reshape(1, D)

    zero = jnp.zeros((N, DH), jnp.float32)
    for l in range(NCONV):
        wd_l = jnp.concatenate([Wf[l][:D], Ws[l][:D]], axis=1)
        wsrc_l = jnp.concatenate([Wf[l][D:2 * D], Ws[l][D:2 * D]], axis=1)
        h, td, ts = _affine_proj(hn, g_l, b_l, mu, den, wd_l, wsrc_l)
        gd, gs = _gather2(td, ts, dst, src)
        msg = _edge_compute(gd, gs, edge_attr, W_ep, b_ep.reshape(1, -1),
                            Wf[l][2 * D:], bf[l].reshape(1, D),
                            Ws[l][2 * D:], bs[l].reshape(1, D))
        agg = _scatter_add(msg, dst, zero)
        hn, s1 = _add_stats(h, agg)
        mu, den = _bn_stats(hn, s1)
        g_l, b_l = bn_g[l].reshape(1, D), bn_b[l].reshape(1, D)
    h = _affine(hn, g_l, b_l, mu, den)

    wo = jnp.concatenate([W_sh, W_bk, W_yg], axis=1)
    bo = jnp.stack([b_sh[0], b_bk[0], b_yg[0]]).reshape(1, 3)
    out3 = _pool_mlp(h, batch.reshape(N // BN_BLK, 1, BN_BLK),
                     W1, b1.reshape(1, -1), W2, b2.reshape(1, -1), wo, bo)
    return out3[:, 0], out3[:, 1], out3[:, 2]


# speedup vs baseline: 3.4364x; 1.0439x over previous
import functools

import jax
import jax.numpy as jnp
from jax import lax
from jax.experimental import pallas as pl
from jax.experimental.pallas import tpu as pltpu
from jax.experimental.pallas import tpu_sc as plsc

N = 50000
E = 800000
G = 256
D = 64
DH = 32
NCONV = 3

NC = 2
NS = 16
NW = NC * NS

PER_W = E // NW
CG = 200
PER_S = E // NS
CS = 200

BN_BLK = 2000
BE_BLK = 4000

_MESH = plsc.VectorSubcoreMesh(
    core_axis_name="c", subcore_axis_name="s", num_cores=NC, num_subcores=NS)


@functools.partial(
    pl.kernel,
    out_type=[jax.ShapeDtypeStruct((E, 2 * D), jnp.float32),
              jax.ShapeDtypeStruct((E, 2 * D), jnp.float32)],
    mesh=_MESH,
    scratch_types=[pltpu.VMEM((CG,), jnp.int32),
                   pltpu.VMEM((CG,), jnp.int32),
                   pltpu.VMEM((CG,), jnp.int32),
                   pltpu.VMEM((CG,), jnp.int32),
                   pltpu.VMEM((CG, 2 * D), jnp.float32),
                   pltpu.VMEM((CG, 2 * D), jnp.float32),
                   pltpu.VMEM((CG, 2 * D), jnp.float32),
                   pltpu.VMEM((CG, 2 * D), jnp.float32),
                   pltpu.SemaphoreType.DMA((NC,)),
                   pltpu.SemaphoreType.DMA((NC,))],
    compiler_params=pltpu.CompilerParams(use_tc_tiling_on_sc=True),
)
def _gather2(td_hbm, ts_hbm, dst_hbm, src_hbm, gd_hbm, gs_hbm,
             di_v0, di_v1, si_v0, si_v1, dr_v0, dr_v1, sr_v0, sr_v1,
             sem_d, sem_s):
    di_vs, si_vs = (di_v0, di_v1), (si_v0, si_v1)
    dr_vs, sr_vs = (dr_v0, dr_v1), (sr_v0, sr_v1)
    wid = lax.axis_index("s") * NC + lax.axis_index("c")
    base = wid * PER_W

    def fire(k, slot):
        b = pl.multiple_of(base + k * CG, 8)
        pltpu.sync_copy(dst_hbm.at[pl.ds(b, CG)], di_vs[slot])
        pltpu.sync_copy(src_hbm.at[pl.ds(b, CG)], si_vs[slot])
        pltpu.async_copy(td_hbm.at[di_vs[slot]], dr_vs[slot],
                         sem_d.at[slot])
        pltpu.async_copy(ts_hbm.at[si_vs[slot]], sr_vs[slot],
                         sem_s.at[slot])

    def drain(k, slot):
        b = pl.multiple_of(base + k * CG, 8)
        pltpu.make_async_copy(td_hbm.at[di_vs[slot]], dr_vs[slot],
                              sem_d.at[slot]).wait()
        pltpu.make_async_copy(ts_hbm.at[si_vs[slot]], sr_vs[slot],
                              sem_s.at[slot]).wait()
        pltpu.sync_copy(dr_vs[slot], gd_hbm.at[pl.ds(b, CG)])
        pltpu.sync_copy(sr_vs[slot], gs_hbm.at[pl.ds(b, CG)])

    fire(0, 0)

    @pl.loop(0, (PER_W // CG - 1) // 2)
    def _(it):
        k0 = 1 + 2 * it
        fire(k0, 1)
        drain(k0 - 1, 0)
        fire(k0 + 1, 0)
        drain(k0, 1)

    drain(PER_W // CG - 1, 0)


@functools.partial(
    pl.kernel,
    out_type=jax.ShapeDtypeStruct((NC, N, DH), jnp.float32),
    mesh=_MESH,
    scratch_types=[pltpu.VMEM((CS,), jnp.int32),
                   pltpu.VMEM((CS,), jnp.int32),
                   pltpu.VMEM((CS, DH), jnp.float32),
                   pltpu.VMEM((CS, DH), jnp.float32),
                   pltpu.VMEM_SHARED((N, DH), jnp.float32),
                   pltpu.SemaphoreType.DMA((NC,)),
                   pltpu.SemaphoreType.DMA((NC,))],
    compiler_params=pltpu.CompilerParams(use_tc_tiling_on_sc=False),
)
def _scatter_add(msg_hbm, dst_hbm, zero_hbm, agg_hbm, idx_v0, idx_v1,
                 m_v0, m_v1, acc_sh, sem_i, sem_m):
    idx_vs = (idx_v0, idx_v1)
    m_vs = (m_v0, m_v1)
    c = lax.axis_index("c")
    s = lax.axis_index("s")

    @pl.when(s == 0)
    def _():
        pltpu.sync_copy(zero_hbm, acc_sh)

    plsc.subcore_barrier()
    base = s * PER_S

    def fire(k, slot):
        b = pl.multiple_of(base + k * CS, 8)
        pltpu.async_copy(dst_hbm.at[pl.ds(b, CS)], idx_vs[slot],
                         sem_i.at[slot])
        pltpu.async_copy(msg_hbm.at[pl.ds(b, CS), pl.ds(DH * c, DH)],
                         m_vs[slot], sem_m.at[slot])

    def add(k, slot):
        b = pl.multiple_of(base + k * CS, 8)
        pltpu.make_async_copy(dst_hbm.at[pl.ds(b, CS)], idx_vs[slot],
                              sem_i.at[slot]).wait()
        pltpu.make_async_copy(msg_hbm.at[pl.ds(b, CS), pl.ds(DH * c, DH)],
                              m_vs[slot], sem_m.at[slot]).wait()
        pltpu.sync_copy(m_vs[slot], acc_sh.at[idx_vs[slot]], add=True)

    fire(0, 0)

    @pl.loop(0, PER_S // CS // 2)
    def _(it):
        k0 = 2 * it

        @pl.when(k0 + 1 < PER_S // CS)
        def _():
            fire(k0 + 1, 1)

        add(k0, 0)

        @pl.when(k0 + 2 < PER_S // CS)
        def _():
            fire(k0 + 2, 0)

        @pl.when(k0 + 1 < PER_S // CS)
        def _():
            add(k0 + 1, 1)

    plsc.subcore_barrier()
    n_chunks = N // 400

    @pl.loop(0, (n_chunks + NS - 1) // NS)
    def _(k):
        chunk = s + k * NS

        @pl.when(chunk < n_chunks)
        def _():
            r0 = pl.multiple_of(chunk * 400, 8)
            pltpu.sync_copy(acc_sh.at[pl.ds(r0, 400)],
                            agg_hbm.at[c, pl.ds(r0, 400)])


def _node_proj_body(x_ref, w_ref, b_ref, h_ref, st_ref):
    h = jax.nn.softplus(
        jnp.dot(x_ref[...], w_ref[...], preferred_element_type=jnp.float32)
        + b_ref[...])
    h_ref[...] = h
    contrib = jnp.sum(h, axis=0, keepdims=True)

    @pl.when(pl.program_id(0) == 0)
    def _():
        st_ref[...] = contrib

    @pl.when(pl.program_id(0) != 0)
    def _():
        st_ref[...] += contrib


def _node_proj(x, w, b):
    nb = N // BN_BLK
    return pl.pallas_call(
        _node_proj_body,
        grid=(nb,),
        in_specs=[pl.BlockSpec((BN_BLK, x.shape[1]), lambda i: (i, 0)),
                  pl.BlockSpec(w.shape, lambda i: (0, 0)),
                  pl.BlockSpec(b.shape, lambda i: (0, 0))],
        out_specs=[pl.BlockSpec((BN_BLK, D), lambda i: (i, 0)),
                   pl.BlockSpec((1, D), lambda i: (0, 0))],
        out_shape=[jax.ShapeDtypeStruct((N, D), jnp.float32),
                   jax.ShapeDtypeStruct((1, D), jnp.float32)],
    )(x, w, b)


def _add_stats_body(h_ref, agg_ref, hn_ref, st_ref):
    a = agg_ref[...]
    hn = h_ref[...] + jnp.concatenate([a[0], a[1]], axis=-1)
    hn_ref[...] = hn
    contrib = jnp.sum(hn, axis=0, keepdims=True)

    @pl.when(pl.program_id(0) == 0)
    def _():
        st_ref[...] = contrib

    @pl.when(pl.program_id(0) != 0)
    def _():
        st_ref[...] += contrib


def _add_stats(h, agg):
    nb = N // BN_BLK
    return pl.pallas_call(
        _add_stats_body,
        grid=(nb,),
        in_specs=[pl.BlockSpec((BN_BLK, D), lambda i: (i, 0)),
                  pl.BlockSpec((NC, BN_BLK, DH), lambda i: (0, i, 0))],
        out_specs=[pl.BlockSpec((BN_BLK, D), lambda i: (i, 0)),
                   pl.BlockSpec((1, D), lambda i: (0, 0))],
        out_shape=[jax.ShapeDtypeStruct((N, D), jnp.float32),
                   jax.ShapeDtypeStruct((1, D), jnp.float32)],
    )(h, agg)


def _var_body(h_ref, mu_ref, st_ref):
    d = h_ref[...] - mu_ref[...]
    contrib = jnp.sum(d * d, axis=0, keepdims=True)

    @pl.when(pl.program_id(0) == 0)
    def _():
        st_ref[...] = contrib

    @pl.when(pl.program_id(0) != 0)
    def _():
        st_ref[...] += contrib


def _var_pass(h, mu):
    nb = N // BN_BLK
    return pl.pallas_call(
        _var_body,
        grid=(nb,),
        in_specs=[pl.BlockSpec((BN_BLK, D), lambda i: (i, 0)),
                  pl.BlockSpec((1, D), lambda i: (0, 0))],
        out_specs=pl.BlockSpec((1, D), lambda i: (0, 0)),
        out_shape=jax.ShapeDtypeStruct((1, D), jnp.float32),
    )(h, mu)


def _affine_proj_body(h_ref, g_ref, b_ref, mu_ref, den_ref, wd_ref, ws_ref,
                      o_ref, td_ref, ts_ref):
    hh = g_ref[...] * (h_ref[...] - mu_ref[...]) / den_ref[...] + b_ref[...]
    o_ref[...] = hh
    td_ref[...] = jnp.dot(hh, wd_ref[...], preferred_element_type=jnp.float32)
    ts_ref[...] = jnp.dot(hh, ws_ref[...], preferred_element_type=jnp.float32)


def _affine_proj(h, g, b, mu, den, wd, ws):
    nb = N // BN_BLK
    return pl.pallas_call(
        _affine_proj_body,
        grid=(nb,),
        in_specs=[pl.BlockSpec((BN_BLK, D), lambda i: (i, 0)),
                  pl.BlockSpec((1, D), lambda i: (0, 0)),
                  pl.BlockSpec((1, D), lambda i: (0, 0)),
                  pl.BlockSpec((1, D), lambda i: (0, 0)),
                  pl.BlockSpec((1, D), lambda i: (0, 0)),
                  pl.BlockSpec(wd.shape, lambda i: (0, 0)),
                  pl.BlockSpec(ws.shape, lambda i: (0, 0))],
        out_specs=[pl.BlockSpec((BN_BLK, D), lambda i: (i, 0)),
                   pl.BlockSpec((BN_BLK, 2 * D), lambda i: (i, 0)),
                   pl.BlockSpec((BN_BLK, 2 * D), lambda i: (i, 0))],
        out_shape=[jax.ShapeDtypeStruct((N, D), jnp.float32),
                   jax.ShapeDtypeStruct((N, 2 * D), jnp.float32),
                   jax.ShapeDtypeStruct((N, 2 * D), jnp.float32)],
        compiler_params=pltpu.CompilerParams(
            dimension_semantics=("parallel",)),
    )(h, g, b, mu, den, wd, ws)


def _affine_body(h_ref, g_ref, b_ref, mu_ref, den_ref, o_ref):
    o_ref[...] = (g_ref[...] * (h_ref[...] - mu_ref[...]) / den_ref[...]
                  + b_ref[...])


def _affine(h, g, b, mu, den):
    nb = N // BN_BLK
    return pl.pallas_call(
        _affine_body,
        grid=(nb,),
        in_specs=[pl.BlockSpec((BN_BLK, D), lambda i: (i, 0)),
                  pl.BlockSpec((1, D), lambda i: (0, 0)),
                  pl.BlockSpec((1, D), lambda i: (0, 0)),
                  pl.BlockSpec((1, D), lambda i: (0, 0)),
                  pl.BlockSpec((1, D), lambda i: (0, 0))],
        out_specs=pl.BlockSpec((BN_BLK, D), lambda i: (i, 0)),
        out_shape=jax.ShapeDtypeStruct((N, D), jnp.float32),
    )(h, g, b, mu, den)


def _edge_body(gd_ref, gs_ref, ea0_ref, ea1_ref, wept_ref, bept_ref,
               wfe_ref, bf_ref, wse_ref, bs_ref, msg_ref):
    r = lambda v: v.astype(jnp.bfloat16).astype(jnp.float32)
    a0 = r(ea0_ref[0])
    a1 = r(ea1_ref[0])
    wept = wept_ref[...]
    e_t = jax.nn.softplus(r(wept[:, 0:1]) * a0 + r(wept[:, 1:2]) * a1
                          + bept_ref[...])
    g = gd_ref[...] + gs_ref[...]
    dotT = lambda et, w: lax.dot_general(
        et, w, dimension_numbers=(((0,), (0,)), ((), ())),
        preferred_element_type=jnp.float32)
    zf = g[:, :D] + dotT(e_t, wfe_ref[...]) + bf_ref[...]
    zs = g[:, D:] + dotT(e_t, wse_ref[...]) + bs_ref[...]
    m = jax.nn.sigmoid(zf) * jax.nn.softplus(zs)
    msg_ref[...] = jnp.concatenate([m, m], axis=-1)


def _edge_compute(gd, gs, ea0, ea1, wept, bept, wfe, bf_l, wse, bs_l):
    nb = E // BE_BLK
    return pl.pallas_call(
        _edge_body,
        grid=(nb,),
        in_specs=[pl.BlockSpec((BE_BLK, 2 * D), lambda i: (i, 0)),
                  pl.BlockSpec((BE_BLK, 2 * D), lambda i: (i, 0)),
                  pl.BlockSpec((1, 1, BE_BLK), lambda i: (i, 0, 0)),
                  pl.BlockSpec((1, 1, BE_BLK), lambda i: (i, 0, 0)),
                  pl.BlockSpec(wept.shape, lambda i: (0, 0)),
                  pl.BlockSpec(bept.shape, lambda i: (0, 0)),
                  pl.BlockSpec(wfe.shape, lambda i: (0, 0)),
                  pl.BlockSpec((1, D), lambda i: (0, 0)),
                  pl.BlockSpec(wse.shape, lambda i: (0, 0)),
                  pl.BlockSpec((1, D), lambda i: (0, 0))],
        out_specs=pl.BlockSpec((BE_BLK, 2 * D), lambda i: (i, 0)),
        out_shape=jax.ShapeDtypeStruct((E, 2 * D), jnp.float32),
        compiler_params=pltpu.CompilerParams(
            dimension_semantics=("parallel",)),
    )(gd, gs, ea0, ea1, wept, bept, wfe, bf_l, wse, bs_l)


def _pool_body(h_ref, bat_ref, w1_ref, b1_ref, w2_ref, b2_ref,
               wo_ref, bo_ref, o_ref, acc_ref):
    bids = bat_ref[0, 0, :]
    oneh_t = (lax.broadcasted_iota(jnp.int32, (G, BN_BLK), 0)
              == bids[None, :]).astype(jnp.float32)
    h = h_ref[...]
    hb = jnp.concatenate([h, jnp.ones((BN_BLK, 1), jnp.float32)], axis=-1)
    contrib = jnp.dot(oneh_t, hb, preferred_element_type=jnp.float32,
                      precision=lax.Precision.HIGHEST)

    @pl.when(pl.program_id(0) == 0)
    def _():
        acc_ref[...] = contrib

    @pl.when(pl.program_id(0) != 0)
    def _():
        acc_ref[...] += contrib

    @pl.when(pl.program_id(0) == pl.num_programs(0) - 1)
    def _():
        acc = acc_ref[...]
        pooled = acc[:, :D] / jnp.maximum(acc[:, D:D + 1], 1.0)
        f = jax.nn.softplus(
            jnp.dot(pooled, w1_ref[...], preferred_element_type=jnp.float32)
            + b1_ref[...])
        f = jax.nn.softplus(
            jnp.dot(f, w2_ref[...], preferred_element_type=jnp.float32)
            + b2_ref[...])
        o_ref[...] = (jnp.dot(f, wo_ref[...],
                              preferred_element_type=jnp.float32)
                      + bo_ref[...])


def _pool_mlp(h, bat3, w1, b1, w2, b2, wo, bo):
    nb = N // BN_BLK
    hdim = w1.shape[1]
    return pl.pallas_call(
        _pool_body,
        grid=(nb,),
        in_specs=[pl.BlockSpec((BN_BLK, D), lambda i: (i, 0)),
                  pl.BlockSpec((1, 1, BN_BLK), lambda i: (i, 0, 0)),
                  pl.BlockSpec(w1.shape, lambda i: (0, 0)),
                  pl.BlockSpec(b1.shape, lambda i: (0, 0)),
                  pl.BlockSpec(w2.shape, lambda i: (0, 0)),
                  pl.BlockSpec(b2.shape, lambda i: (0, 0)),
                  pl.BlockSpec(wo.shape, lambda i: (0, 0)),
                  pl.BlockSpec(bo.shape, lambda i: (0, 0))],
        out_specs=pl.BlockSpec((G, wo.shape[1]), lambda i: (0, 0)),
        out_shape=jax.ShapeDtypeStruct((G, wo.shape[1]), jnp.float32),
        scratch_shapes=[pltpu.VMEM((G, D + 1), jnp.float32)],
    )(h, bat3, w1, b1, w2, b2, wo, bo)


def _bn_stats(h, s1):
    mu = s1 / N
    var = _var_pass(h, mu) / N
    den = jnp.sqrt(var + 1e-5)
    return mu, den


def kernel(x, edge_index, edge_attr, batch, W_np, b_np, bn0_g, bn0_b,
           W_ep, b_ep, Wf, bf, Ws, bs, bn_g, bn_b, W1, b1, W2, b2,
           W_sh, b_sh, W_bk, b_bk, W_yg, b_yg):
    src = edge_index[0]
    dst = edge_index[1]

    hn, s1 = _node_proj(x, W_np, b_np.reshape(1, D))
    mu, den = _bn_stats(hn, s1)
    g_l, b_l = bn0_g.reshape(1, D), bn0_b.reshape(1, D)

    zero = jnp.zeros((N, DH), jnp.float32)
    nbe = E // BE_BLK
    ea0 = edge_attr[:, 0].reshape(nbe, 1, BE_BLK)
    ea1 = edge_attr[:, 1].reshape(nbe, 1, BE_BLK)
    wept = W_ep.T
    bept = b_ep.reshape(-1, 1)
    for l in range(NCONV):
        wd_l = jnp.concatenate([Wf[l][:D], Ws[l][:D]], axis=1)
        wsrc_l = jnp.concatenate([Wf[l][D:2 * D], Ws[l][D:2 * D]], axis=1)
        h, td, ts = _affine_proj(hn, g_l, b_l, mu, den, wd_l, wsrc_l)
        gd, gs = _gather2(td, ts, dst, src)
        msg = _edge_compute(gd, gs, ea0, ea1, wept, bept,
                            Wf[l][2 * D:], bf[l].reshape(1, D),
                            Ws[l][2 * D:], bs[l].reshape(1, D))
        agg = _scatter_add(msg, dst, zero)
        hn, s1 = _add_stats(h, agg)
        mu, den = _bn_stats(hn, s1)
        g_l, b_l = bn_g[l].reshape(1, D), bn_b[l].reshape(1, D)
    h = _affine(hn, g_l, b_l, mu, den)

    wo = jnp.concatenate([W_sh, W_bk, W_yg], axis=1)
    bo = jnp.stack([b_sh[0], b_bk[0], b_yg[0]]).reshape(1, 3)
    out3 = _pool_mlp(h, batch.reshape(N // BN_BLK, 1, BN_BLK),
                     W1, b1.reshape(1, -1), W2, b2.reshape(1, -1), wo, bo)
    return out3[:, 0], out3[:, 1], out3[:, 2]
